# baseline (speedup 1.0000x reference)
import functools

import jax
import jax.numpy as jnp
from jax import lax
from jax.experimental import pallas as pl
from jax.experimental.pallas import tpu as pltpu
from jax.experimental.pallas import tpu_sc as plsc

N = 10000
E = 160000
D_IN = 256
D_HID = 512
D_OUT = 256

C = 128
B = 128
NTILES = 16
NCORES = 2
NACC = 10112
PAD_DST = N + 8

NB_MAIN = 80
E_MAIN = NTILES * NB_MAIN * B

NB_DEG = 40
E_DEG = NCORES * NTILES * NB_DEG * B

NPAD = NACC
ROWS_PER_TILE = NPAD // NTILES


def _sc_mesh():
    return plsc.VectorSubcoreMesh(core_axis_name="c", subcore_axis_name="s")


def _deg_body(dst_hbm, ones_hbm, zeros_hbm, out_hbm, idx_v, ones_v, sem, acc):
    c = lax.axis_index("c")
    s = lax.axis_index("s")
    pltpu.sync_copy(dst_hbm.at[c].at[s], idx_v)
    pltpu.sync_copy(ones_hbm, ones_v)
    z0 = s * (NACC // NTILES)
    pltpu.sync_copy(zeros_hbm, acc.at[pl.ds(z0, NACC // NTILES)])
    plsc.subcore_barrier()

    def step(j, carry):
        pltpu.sync_copy(ones_v, acc.at[idx_v.at[j]], add=True)
        return carry

    lax.fori_loop(0, NB_DEG, step, 0)
    plsc.subcore_barrier()
    r0 = s * (NACC // NTILES)
    pltpu.sync_copy(acc.at[pl.ds(r0, NACC // NTILES)],
                    out_hbm.at[c].at[pl.ds(r0, NACC // NTILES)])


def _make_deg_kernel():
    return functools.partial(
        pl.kernel,
        out_type=jax.ShapeDtypeStruct((NCORES, NACC, C), jnp.float32),
        mesh=_sc_mesh(),
        scratch_types=[
            pltpu.VMEM((NB_DEG, B), jnp.int32),
            pltpu.VMEM((B, C), jnp.float32),
            pltpu.SemaphoreType.DMA,
            pltpu.VMEM_SHARED((NACC, C), jnp.float32),
        ],
    )(_deg_body)


def _make_scatter_body(chunks_per_core):
    def body(y_hbm, src_hbm, dst_hbm, out_hbm, src_v, dst_v, buf0, sg0, acc):
        c = lax.axis_index("c")
        s = lax.axis_index("s")
        r0 = s * ROWS_PER_TILE
        pltpu.sync_copy(src_hbm.at[s], src_v)
        pltpu.sync_copy(dst_hbm.at[s], dst_v)
        for cc_local in range(chunks_per_core):
            cc = c * chunks_per_core + cc_local
            yc = y_hbm.at[cc]
            pltpu.sync_copy(yc.at[pl.ds(r0, ROWS_PER_TILE)],
                            acc.at[pl.ds(r0, ROWS_PER_TILE)])
            plsc.subcore_barrier()

            def step(j, carry):
                pltpu.async_copy(yc.at[src_v.at[j]], buf0, sg0).wait()
                pltpu.sync_copy(buf0, acc.at[dst_v.at[j]], add=True)
                return carry

            lax.fori_loop(0, NB_MAIN, step, 0)
            plsc.subcore_barrier()
            pltpu.sync_copy(acc.at[pl.ds(r0, ROWS_PER_TILE)],
                            out_hbm.at[cc].at[pl.ds(r0, ROWS_PER_TILE)])
            plsc.subcore_barrier()

    return body


def _make_scatter_kernel(n_chunks):
    chunks_per_core = n_chunks // NCORES
    return functools.partial(
        pl.kernel,
        out_type=jax.ShapeDtypeStruct((n_chunks, NPAD, C), jnp.float32),
        mesh=_sc_mesh(),
        scratch_types=[
            pltpu.VMEM((NB_MAIN, B), jnp.int32),
            pltpu.VMEM((NB_MAIN, B), jnp.int32),
            pltpu.VMEM((B, C), jnp.float32),
            pltpu.SemaphoreType.DMA,
            pltpu.VMEM_SHARED((NACC, C), jnp.float32),
        ],
    )(_make_scatter_body(chunks_per_core))


RB = NPAD // 16


def _mm1_body(deg_ref, x_ref, w_ref, o_ref):
    deg = deg_ref[0, :, 0] + deg_ref[1, :, 0] + 1.0
    dinv = lax.rsqrt(deg)[:, None]
    o_ref[0] = dinv * jnp.dot(x_ref[...], w_ref[...],
                              preferred_element_type=jnp.float32)


def _tc_mm1(deg2, x, w1):
    n_chunks = w1.shape[1] // C
    return pl.pallas_call(
        _mm1_body,
        grid=(NPAD // RB, n_chunks),
        in_specs=[
            pl.BlockSpec((NCORES, RB, C), lambda i, j: (0, i, 0)),
            pl.BlockSpec((RB, D_IN), lambda i, j: (i, 0)),
            pl.BlockSpec((D_IN, C), lambda i, j: (0, j)),
        ],
        out_specs=pl.BlockSpec((1, RB, C), lambda i, j: (j, i, 0)),
        out_shape=jax.ShapeDtypeStruct((n_chunks, NPAD, C), jnp.float32),
    )(deg2, x, w1)


def _mm2_body(deg_ref, s1_ref, b1_ref, w_ref, o_ref):
    deg = deg_ref[0, :, 0] + deg_ref[1, :, 0] + 1.0
    dinv = lax.rsqrt(deg)[:, None]
    s1 = jnp.concatenate([s1_ref[k] for k in range(4)], axis=1)
    h = jax.nn.relu(dinv * s1 + b1_ref[0][None, :])
    o_ref[0] = dinv * jnp.dot(h, w_ref[...], preferred_element_type=jnp.float32)


def _tc_mm2(deg2, s1, b1, w2):
    n1 = s1.shape[0]
    n_chunks = w2.shape[1] // C
    return pl.pallas_call(
        _mm2_body,
        grid=(NPAD // RB, n_chunks),
        in_specs=[
            pl.BlockSpec((NCORES, RB, C), lambda i, j: (0, i, 0)),
            pl.BlockSpec((n1, RB, C), lambda i, j: (0, i, 0)),
            pl.BlockSpec((1, D_HID), lambda i, j: (0, 0)),
            pl.BlockSpec((D_HID, C), lambda i, j: (0, j)),
        ],
        out_specs=pl.BlockSpec((1, RB, C), lambda i, j: (j, i, 0)),
        out_shape=jax.ShapeDtypeStruct((n_chunks, NPAD, C), jnp.float32),
    )(deg2, s1, b1, w2)


def _fin_body(deg_ref, s2_ref, b2_ref, o_ref):
    deg = deg_ref[0, :, 0] + deg_ref[1, :, 0] + 1.0
    dinv = lax.rsqrt(deg)[:, None]
    s2 = jnp.concatenate([s2_ref[k] for k in range(2)], axis=1)
    o_ref[...] = dinv * s2 + b2_ref[0][None, :]


def _tc_final(deg2, s2, b2):
    n2 = s2.shape[0]
    return pl.pallas_call(
        _fin_body,
        grid=(NPAD // RB,),
        in_specs=[
            pl.BlockSpec((NCORES, RB, C), lambda i: (0, i, 0)),
            pl.BlockSpec((n2, RB, C), lambda i: (0, i, 0)),
            pl.BlockSpec((1, D_OUT), lambda i: (0, 0)),
        ],
        out_specs=pl.BlockSpec((RB, D_OUT), lambda i: (i, 0)),
        out_shape=jax.ShapeDtypeStruct((NPAD, D_OUT), jnp.float32),
    )(deg2, s2, b2)


def _pad_indices(src, dst):
    pad = E_MAIN - E
    src_p = jnp.concatenate([src, jnp.zeros((pad,), jnp.int32)])
    dst_p = jnp.concatenate([dst, jnp.full((pad,), PAD_DST, jnp.int32)])
    return (src_p.reshape(NTILES, NB_MAIN, B),
            dst_p.reshape(NTILES, NB_MAIN, B))


def _pad_deg_indices(dst):
    pad = E_DEG - E
    dst_p = jnp.concatenate([dst, jnp.full((pad,), PAD_DST, jnp.int32)])
    return dst_p.reshape(NCORES, NTILES, NB_DEG, B)


@jax.jit
def kernel(x, edge_index, W1, b1, W2, b2):
    src = edge_index[0].astype(jnp.int32)
    dst = edge_index[1].astype(jnp.int32)
    src_m, dst_m = _pad_indices(src, dst)
    dst_d = _pad_deg_indices(dst)

    ones = jnp.ones((B, C), jnp.float32)
    zeros = jnp.zeros((NACC // NTILES, C), jnp.float32)

    deg2 = _make_deg_kernel()(dst_d, ones, zeros)

    x_p = jnp.zeros((NPAD, D_IN), jnp.float32).at[:N].set(x)
    y1 = _tc_mm1(deg2, x_p, W1)
    s1 = _make_scatter_kernel(D_HID // C)(y1, src_m, dst_m)
    y2 = _tc_mm2(deg2, s1, b1.reshape(1, D_HID), W2)
    s2 = _make_scatter_kernel(D_OUT // C)(y2, src_m, dst_m)
    out = _tc_final(deg2, s2, b2.reshape(1, D_OUT))
    return out[:N]


# speedup vs baseline: 1.7423x; 1.7423x over previous
import functools

import jax
import jax.numpy as jnp
from jax import lax
from jax.experimental import pallas as pl
from jax.experimental.pallas import tpu as pltpu
from jax.experimental.pallas import tpu_sc as plsc

N = 10000
E = 160000
D_IN = 256
D_HID = 512
D_OUT = 256

C = 128
B = 128
NTILES = 16
NCORES = 2
NACC = 10112
PAD_DST = N + 8

NB_MAIN = 80
E_MAIN = NTILES * NB_MAIN * B

NB_DEG = 40
E_DEG = NCORES * NTILES * NB_DEG * B

NPAD = NACC
ROWS_PER_TILE = NPAD // NTILES


def _sc_mesh():
    return plsc.VectorSubcoreMesh(core_axis_name="c", subcore_axis_name="s")


def _deg_body(dst_hbm, ones_hbm, zeros_hbm, out_hbm, idx_v, ones_v, sem, acc):
    c = lax.axis_index("c")
    s = lax.axis_index("s")
    pltpu.sync_copy(dst_hbm.at[c].at[s], idx_v)
    pltpu.sync_copy(ones_hbm, ones_v)
    z0 = s * (NACC // NTILES)
    pltpu.sync_copy(zeros_hbm, acc.at[pl.ds(z0, NACC // NTILES)])
    plsc.subcore_barrier()

    def step(j, carry):
        pltpu.sync_copy(ones_v, acc.at[idx_v.at[j]], add=True)
        return carry

    lax.fori_loop(0, NB_DEG, step, 0)
    plsc.subcore_barrier()
    r0 = s * (NACC // NTILES)
    pltpu.sync_copy(acc.at[pl.ds(r0, NACC // NTILES)],
                    out_hbm.at[c].at[pl.ds(r0, NACC // NTILES)])


def _make_deg_kernel():
    return functools.partial(
        pl.kernel,
        out_type=jax.ShapeDtypeStruct((NCORES, NACC, C), jnp.float32),
        mesh=_sc_mesh(),
        scratch_types=[
            pltpu.VMEM((NB_DEG, B), jnp.int32),
            pltpu.VMEM((B, C), jnp.float32),
            pltpu.SemaphoreType.DMA,
            pltpu.VMEM_SHARED((NACC, C), jnp.float32),
        ],
    )(_deg_body)


def _make_scatter_body(chunks_per_core):
    def body(y_hbm, src_hbm, dst_hbm, out_hbm, src_v, dst_v, buf0, sg0, acc):
        c = lax.axis_index("c")
        s = lax.axis_index("s")
        r0 = s * ROWS_PER_TILE
        pltpu.sync_copy(src_hbm.at[s], src_v)
        pltpu.sync_copy(dst_hbm.at[s], dst_v)
        for cc_local in range(chunks_per_core):
            cc = c * chunks_per_core + cc_local
            yc = y_hbm.at[cc]
            pltpu.sync_copy(yc.at[pl.ds(r0, ROWS_PER_TILE)],
                            acc.at[pl.ds(r0, ROWS_PER_TILE)])
            plsc.subcore_barrier()

            def step(j, carry):
                pltpu.async_copy(yc.at[src_v.at[j]], buf0, sg0).wait()
                pltpu.sync_copy(buf0, acc.at[dst_v.at[j]], add=True)
                return carry

            lax.fori_loop(0, NB_MAIN, step, 0)
            plsc.subcore_barrier()
            pltpu.sync_copy(acc.at[pl.ds(r0, ROWS_PER_TILE)],
                            out_hbm.at[cc].at[pl.ds(r0, ROWS_PER_TILE)])
            plsc.subcore_barrier()

    return body


def _make_scatter_kernel(n_chunks):
    chunks_per_core = n_chunks // NCORES
    return functools.partial(
        pl.kernel,
        out_type=jax.ShapeDtypeStruct((n_chunks, NPAD, C), jnp.float32),
        mesh=_sc_mesh(),
        scratch_types=[
            pltpu.VMEM((NB_MAIN, B), jnp.int32),
            pltpu.VMEM((NB_MAIN, B), jnp.int32),
            pltpu.VMEM((B, C), jnp.float32),
            pltpu.SemaphoreType.DMA,
            pltpu.VMEM_SHARED((NACC, C), jnp.float32),
        ],
    )(_make_scatter_body(chunks_per_core))


RB = NPAD // 16


def _mm1_body(deg_ref, x_ref, w_ref, o_ref):
    deg = deg_ref[0, :, 0] + deg_ref[1, :, 0] + 1.0
    dinv = lax.rsqrt(deg)[:, None]
    o_ref[0] = dinv * jnp.dot(x_ref[...], w_ref[...],
                              preferred_element_type=jnp.float32)


def _tc_mm1(deg2, x, w1):
    n_chunks = w1.shape[1] // C
    return pl.pallas_call(
        _mm1_body,
        grid=(NPAD // RB, n_chunks),
        in_specs=[
            pl.BlockSpec((NCORES, RB, C), lambda i, j: (0, i, 0)),
            pl.BlockSpec((RB, D_IN), lambda i, j: (i, 0)),
            pl.BlockSpec((D_IN, C), lambda i, j: (0, j)),
        ],
        out_specs=pl.BlockSpec((1, RB, C), lambda i, j: (j, i, 0)),
        out_shape=jax.ShapeDtypeStruct((n_chunks, NPAD, C), jnp.float32),
    )(deg2, x, w1)


def _mm2_body(deg_ref, s1_ref, b1_ref, w_ref, o_ref):
    deg = deg_ref[0, :, 0] + deg_ref[1, :, 0] + 1.0
    dinv = lax.rsqrt(deg)[:, None]
    s1 = jnp.concatenate([s1_ref[k] for k in range(4)], axis=1)
    h = jax.nn.relu(dinv * s1 + b1_ref[0][None, :])
    o_ref[0] = dinv * jnp.dot(h, w_ref[...], preferred_element_type=jnp.float32)


def _tc_mm2(deg2, s1, b1, w2):
    n1 = s1.shape[0]
    n_chunks = w2.shape[1] // C
    return pl.pallas_call(
        _mm2_body,
        grid=(NPAD // RB, n_chunks),
        in_specs=[
            pl.BlockSpec((NCORES, RB, C), lambda i, j: (0, i, 0)),
            pl.BlockSpec((n1, RB, C), lambda i, j: (0, i, 0)),
            pl.BlockSpec((1, D_HID), lambda i, j: (0, 0)),
            pl.BlockSpec((D_HID, C), lambda i, j: (0, j)),
        ],
        out_specs=pl.BlockSpec((1, RB, C), lambda i, j: (j, i, 0)),
        out_shape=jax.ShapeDtypeStruct((n_chunks, NPAD, C), jnp.float32),
    )(deg2, s1, b1, w2)


def _fin_body(deg_ref, s2_ref, b2_ref, o_ref):
    deg = deg_ref[0, :, 0] + deg_ref[1, :, 0] + 1.0
    dinv = lax.rsqrt(deg)[:, None]
    s2 = jnp.concatenate([s2_ref[k] for k in range(2)], axis=1)
    o_ref[...] = dinv * s2 + b2_ref[0][None, :]


def _tc_final(deg2, s2, b2):
    n2 = s2.shape[0]
    return pl.pallas_call(
        _fin_body,
        grid=(NPAD // RB,),
        in_specs=[
            pl.BlockSpec((NCORES, RB, C), lambda i: (0, i, 0)),
            pl.BlockSpec((n2, RB, C), lambda i: (0, i, 0)),
            pl.BlockSpec((1, D_OUT), lambda i: (0, 0)),
        ],
        out_specs=pl.BlockSpec((RB, D_OUT), lambda i: (i, 0)),
        out_shape=jax.ShapeDtypeStruct((NPAD, D_OUT), jnp.float32),
    )(deg2, s2, b2)


def _pad_scatter_rows(n):
    return N + (jnp.arange(n, dtype=jnp.int32) % (NACC - N))


def _pad_indices(src, dst):
    pad = E_MAIN - E
    src_p = jnp.concatenate([src, jnp.arange(pad, dtype=jnp.int32) % N])
    dst_p = jnp.concatenate([dst, _pad_scatter_rows(pad)])
    return (src_p.reshape(NTILES, NB_MAIN, B),
            dst_p.reshape(NTILES, NB_MAIN, B))


def _pad_deg_indices(dst):
    pad = E_DEG - E
    dst_p = jnp.concatenate([dst, _pad_scatter_rows(pad)])
    return dst_p.reshape(NCORES, NTILES, NB_DEG, B)


@jax.jit
def kernel(x, edge_index, W1, b1, W2, b2):
    src = edge_index[0].astype(jnp.int32)
    dst = edge_index[1].astype(jnp.int32)
    src_m, dst_m = _pad_indices(src, dst)
    dst_d = _pad_deg_indices(dst)

    ones = jnp.ones((B, C), jnp.float32)
    zeros = jnp.zeros((NACC // NTILES, C), jnp.float32)

    deg2 = _make_deg_kernel()(dst_d, ones, zeros)

    x_p = jnp.zeros((NPAD, D_IN), jnp.float32).at[:N].set(x)
    y1 = _tc_mm1(deg2, x_p, W1)
    s1 = _make_scatter_kernel(D_HID // C)(y1, src_m, dst_m)
    y2 = _tc_mm2(deg2, s1, b1.reshape(1, D_HID), W2)
    s2 = _make_scatter_kernel(D_OUT // C)(y2, src_m, dst_m)
    out = _tc_final(deg2, s2, b2.reshape(1, D_OUT))
    return out[:N]


# speedup vs baseline: 2.3931x; 1.3735x over previous
import functools

import jax
import jax.numpy as jnp
from jax import lax
from jax.experimental import pallas as pl
from jax.experimental.pallas import tpu as pltpu
from jax.experimental.pallas import tpu_sc as plsc

N = 10000
E = 160000
D_IN = 256
D_HID = 512
D_OUT = 256

C = 128
B = 128
NTILES = 16
NCORES = 2
NACC = 10112
PAD_DST = N + 8

NB_MAIN = 80
E_MAIN = NTILES * NB_MAIN * B

NB_DEG = 40
E_DEG = NCORES * NTILES * NB_DEG * B

NPAD = NACC
ROWS_PER_TILE = NPAD // NTILES


def _sc_mesh():
    return plsc.VectorSubcoreMesh(core_axis_name="c", subcore_axis_name="s")


def _deg_body(dst_hbm, ones_hbm, zeros_hbm, out_hbm, idx_v, ones_v, sem, acc):
    c = lax.axis_index("c")
    s = lax.axis_index("s")
    pltpu.sync_copy(dst_hbm.at[c].at[s], idx_v)
    pltpu.sync_copy(ones_hbm, ones_v)
    z0 = s * (NACC // NTILES)
    pltpu.sync_copy(zeros_hbm, acc.at[pl.ds(z0, NACC // NTILES)])
    plsc.subcore_barrier()

    def step(j, carry):
        pltpu.sync_copy(ones_v, acc.at[idx_v.at[j]], add=True)
        return carry

    lax.fori_loop(0, NB_DEG, step, 0)
    plsc.subcore_barrier()
    r0 = s * (NACC // NTILES)
    pltpu.sync_copy(acc.at[pl.ds(r0, NACC // NTILES)],
                    out_hbm.at[c].at[pl.ds(r0, NACC // NTILES)])


def _make_deg_kernel():
    return functools.partial(
        pl.kernel,
        out_type=jax.ShapeDtypeStruct((NCORES, NACC, C), jnp.float32),
        mesh=_sc_mesh(),
        scratch_types=[
            pltpu.VMEM((NB_DEG, B), jnp.int32),
            pltpu.VMEM((B, C), jnp.float32),
            pltpu.SemaphoreType.DMA,
            pltpu.VMEM_SHARED((NACC, C), jnp.float32),
        ],
    )(_deg_body)


def _make_scatter_body(chunks_per_core):
    def body(y_hbm, src_hbm, dst_hbm, out_hbm, src_v, dst_v,
             buf0, buf1, sg0, sg1, acc):
        c = lax.axis_index("c")
        s = lax.axis_index("s")
        r0 = s * ROWS_PER_TILE
        HNB = NB_MAIN // 2
        H = HNB // 2
        for cc_local in range(chunks_per_core):
            cc = c * chunks_per_core + cc_local
            yc = y_hbm.at[cc]
            pltpu.sync_copy(yc.at[pl.ds(r0, ROWS_PER_TILE)],
                            acc.at[pl.ds(r0, ROWS_PER_TILE)])
            plsc.subcore_barrier()

            for half in range(2):
                pltpu.sync_copy(src_hbm.at[s].at[pl.ds(half * HNB, HNB)], src_v)
                pltpu.sync_copy(dst_hbm.at[s].at[pl.ds(half * HNB, HNB)], dst_v)
                pltpu.async_copy(yc.at[src_v.at[0]], buf0, sg0)

                def step(p, carry):
                    j0 = 2 * p
                    pltpu.async_copy(yc.at[src_v.at[j0 + 1]], buf1, sg1)
                    pltpu.make_async_copy(yc.at[src_v.at[j0]], buf0, sg0).wait()
                    pltpu.sync_copy(buf0, acc.at[dst_v.at[j0]], add=True)

                    @pl.when(p < H - 1)
                    def _():
                        pltpu.async_copy(yc.at[src_v.at[j0 + 2]], buf0, sg0)

                    pltpu.make_async_copy(yc.at[src_v.at[j0 + 1]],
                                          buf1, sg1).wait()
                    pltpu.sync_copy(buf1, acc.at[dst_v.at[j0 + 1]], add=True)
                    return carry

                lax.fori_loop(0, H, step, 0)
            plsc.subcore_barrier()
            pltpu.sync_copy(acc.at[pl.ds(r0, ROWS_PER_TILE)],
                            out_hbm.at[cc].at[pl.ds(r0, ROWS_PER_TILE)])
            plsc.subcore_barrier()

    return body


def _make_scatter_kernel(n_chunks):
    chunks_per_core = n_chunks // NCORES
    return functools.partial(
        pl.kernel,
        out_type=jax.ShapeDtypeStruct((n_chunks, NPAD, C), jnp.float32),
        mesh=_sc_mesh(),
        scratch_types=[
            pltpu.VMEM((NB_MAIN // 2, B), jnp.int32),
            pltpu.VMEM((NB_MAIN // 2, B), jnp.int32),
            pltpu.VMEM((B, C), jnp.float32),
            pltpu.VMEM((B, C), jnp.float32),
            pltpu.SemaphoreType.DMA,
            pltpu.SemaphoreType.DMA,
            pltpu.VMEM_SHARED((NACC, C), jnp.float32),
        ],
    )(_make_scatter_body(chunks_per_core))


RB = NPAD // 16


def _mm1_body(deg_ref, x_ref, w_ref, o_ref):
    deg = deg_ref[0, :, 0] + deg_ref[1, :, 0] + 1.0
    dinv = lax.rsqrt(deg)[:, None]
    o_ref[0] = dinv * jnp.dot(x_ref[...], w_ref[...],
                              preferred_element_type=jnp.float32)


def _tc_mm1(deg2, x, w1):
    n_chunks = w1.shape[1] // C
    return pl.pallas_call(
        _mm1_body,
        grid=(NPAD // RB, n_chunks),
        in_specs=[
            pl.BlockSpec((NCORES, RB, C), lambda i, j: (0, i, 0)),
            pl.BlockSpec((RB, D_IN), lambda i, j: (i, 0)),
            pl.BlockSpec((D_IN, C), lambda i, j: (0, j)),
        ],
        out_specs=pl.BlockSpec((1, RB, C), lambda i, j: (j, i, 0)),
        out_shape=jax.ShapeDtypeStruct((n_chunks, NPAD, C), jnp.float32),
    )(deg2, x, w1)


def _mm2_body(deg_ref, s1_ref, b1_ref, w_ref, o_ref):
    deg = deg_ref[0, :, 0] + deg_ref[1, :, 0] + 1.0
    dinv = lax.rsqrt(deg)[:, None]
    s1 = jnp.concatenate([s1_ref[k] for k in range(4)], axis=1)
    h = jax.nn.relu(dinv * s1 + b1_ref[0][None, :])
    o_ref[0] = dinv * jnp.dot(h, w_ref[...], preferred_element_type=jnp.float32)


def _tc_mm2(deg2, s1, b1, w2):
    n1 = s1.shape[0]
    n_chunks = w2.shape[1] // C
    return pl.pallas_call(
        _mm2_body,
        grid=(NPAD // RB, n_chunks),
        in_specs=[
            pl.BlockSpec((NCORES, RB, C), lambda i, j: (0, i, 0)),
            pl.BlockSpec((n1, RB, C), lambda i, j: (0, i, 0)),
            pl.BlockSpec((1, D_HID), lambda i, j: (0, 0)),
            pl.BlockSpec((D_HID, C), lambda i, j: (0, j)),
        ],
        out_specs=pl.BlockSpec((1, RB, C), lambda i, j: (j, i, 0)),
        out_shape=jax.ShapeDtypeStruct((n_chunks, NPAD, C), jnp.float32),
    )(deg2, s1, b1, w2)


def _fin_body(deg_ref, s2_ref, b2_ref, o_ref):
    deg = deg_ref[0, :, 0] + deg_ref[1, :, 0] + 1.0
    dinv = lax.rsqrt(deg)[:, None]
    s2 = jnp.concatenate([s2_ref[k] for k in range(2)], axis=1)
    o_ref[...] = dinv * s2 + b2_ref[0][None, :]


def _tc_final(deg2, s2, b2):
    n2 = s2.shape[0]
    return pl.pallas_call(
        _fin_body,
        grid=(NPAD // RB,),
        in_specs=[
            pl.BlockSpec((NCORES, RB, C), lambda i: (0, i, 0)),
            pl.BlockSpec((n2, RB, C), lambda i: (0, i, 0)),
            pl.BlockSpec((1, D_OUT), lambda i: (0, 0)),
        ],
        out_specs=pl.BlockSpec((RB, D_OUT), lambda i: (i, 0)),
        out_shape=jax.ShapeDtypeStruct((NPAD, D_OUT), jnp.float32),
    )(deg2, s2, b2)


def _pad_scatter_rows(n):
    return N + (jnp.arange(n, dtype=jnp.int32) % (NACC - N))


def _pad_indices(src, dst):
    pad = E_MAIN - E
    src_p = jnp.concatenate([src, jnp.arange(pad, dtype=jnp.int32) % N])
    dst_p = jnp.concatenate([dst, _pad_scatter_rows(pad)])
    return (src_p.reshape(NTILES, NB_MAIN, B),
            dst_p.reshape(NTILES, NB_MAIN, B))


def _pad_deg_indices(dst):
    pad = E_DEG - E
    dst_p = jnp.concatenate([dst, _pad_scatter_rows(pad)])
    return dst_p.reshape(NCORES, NTILES, NB_DEG, B)


@jax.jit
def kernel(x, edge_index, W1, b1, W2, b2):
    src = edge_index[0].astype(jnp.int32)
    dst = edge_index[1].astype(jnp.int32)
    src_m, dst_m = _pad_indices(src, dst)
    dst_d = _pad_deg_indices(dst)

    ones = jnp.ones((B, C), jnp.float32)
    zeros = jnp.zeros((NACC // NTILES, C), jnp.float32)

    deg2 = _make_deg_kernel()(dst_d, ones, zeros)

    x_p = jnp.zeros((NPAD, D_IN), jnp.float32).at[:N].set(x)
    y1 = _tc_mm1(deg2, x_p, W1)
    s1 = _make_scatter_kernel(D_HID // C)(y1, src_m, dst_m)
    y2 = _tc_mm2(deg2, s1, b1.reshape(1, D_HID), W2)
    s2 = _make_scatter_kernel(D_OUT // C)(y2, src_m, dst_m)
    out = _tc_final(deg2, s2, b2.reshape(1, D_OUT))
    return out[:N]


# speedup vs baseline: 2.6986x; 1.1277x over previous
import functools

import jax
import jax.numpy as jnp
from jax import lax
from jax.experimental import pallas as pl
from jax.experimental.pallas import tpu as pltpu
from jax.experimental.pallas import tpu_sc as plsc

N = 10000
E = 160000
D_IN = 256
D_HID = 512
D_OUT = 256

C = 128
B = 128
NTILES = 16
NCORES = 2
NACC = 10240
PAD_DST = N + 8

NB_MAIN = 80
E_MAIN = NTILES * NB_MAIN * B

NB_DEG = 40
E_DEG = NCORES * NTILES * NB_DEG * B

NPAD = NACC
ROWS_PER_TILE = NPAD // NTILES


def _sc_mesh():
    return plsc.VectorSubcoreMesh(core_axis_name="c", subcore_axis_name="s")


def _deg_body(dst_hbm, ones_hbm, zeros_hbm, out_hbm, idx_v, ones_v, sem, acc):
    c = lax.axis_index("c")
    s = lax.axis_index("s")
    pltpu.sync_copy(dst_hbm.at[c].at[s], idx_v)
    pltpu.sync_copy(ones_hbm, ones_v)
    z0 = s * (NACC // NTILES)
    pltpu.sync_copy(zeros_hbm, acc.at[pl.ds(z0, NACC // NTILES)])
    plsc.subcore_barrier()

    def step(j, carry):
        pltpu.sync_copy(ones_v, acc.at[idx_v.at[j]], add=True)
        return carry

    lax.fori_loop(0, NB_DEG, step, 0)
    plsc.subcore_barrier()
    r0 = s * (NACC // NTILES)
    pltpu.sync_copy(acc.at[pl.ds(r0, NACC // NTILES)],
                    out_hbm.at[c].at[pl.ds(r0, NACC // NTILES)])


def _make_deg_kernel():
    return functools.partial(
        pl.kernel,
        out_type=jax.ShapeDtypeStruct((NCORES, NACC, C), jnp.float32),
        mesh=_sc_mesh(),
        scratch_types=[
            pltpu.VMEM((NB_DEG, B), jnp.int32),
            pltpu.VMEM((B, C), jnp.float32),
            pltpu.SemaphoreType.DMA,
            pltpu.VMEM_SHARED((NACC, C), jnp.float32),
        ],
    )(_deg_body)


def _make_scatter_body(chunks_per_core):
    def body(y_hbm, src_hbm, dst_hbm, out_hbm, src_v, dst_v,
             buf0, buf1, sg0, sg1, acc):
        c = lax.axis_index("c")
        s = lax.axis_index("s")
        r0 = s * ROWS_PER_TILE
        HNB = NB_MAIN // 2
        H = HNB // 2
        for cc_local in range(chunks_per_core):
            cc = c * chunks_per_core + cc_local
            yc = y_hbm.at[cc]
            pltpu.sync_copy(yc.at[pl.ds(r0, ROWS_PER_TILE)],
                            acc.at[pl.ds(r0, ROWS_PER_TILE)])
            plsc.subcore_barrier()

            for half in range(2):
                pltpu.sync_copy(src_hbm.at[s].at[pl.ds(half * HNB, HNB)], src_v)
                pltpu.sync_copy(dst_hbm.at[s].at[pl.ds(half * HNB, HNB)], dst_v)
                pltpu.async_copy(yc.at[src_v.at[0]], buf0, sg0)

                def step(p, carry):
                    j0 = 2 * p
                    pltpu.async_copy(yc.at[src_v.at[j0 + 1]], buf1, sg1)
                    pltpu.make_async_copy(yc.at[src_v.at[j0]], buf0, sg0).wait()
                    pltpu.sync_copy(buf0, acc.at[dst_v.at[j0]], add=True)

                    @pl.when(p < H - 1)
                    def _():
                        pltpu.async_copy(yc.at[src_v.at[j0 + 2]], buf0, sg0)

                    pltpu.make_async_copy(yc.at[src_v.at[j0 + 1]],
                                          buf1, sg1).wait()
                    pltpu.sync_copy(buf1, acc.at[dst_v.at[j0 + 1]], add=True)
                    return carry

                lax.fori_loop(0, H, step, 0)
            plsc.subcore_barrier()
            pltpu.sync_copy(acc.at[pl.ds(r0, ROWS_PER_TILE)],
                            out_hbm.at[cc].at[pl.ds(r0, ROWS_PER_TILE)])
            plsc.subcore_barrier()

    return body


def _make_scatter_kernel(n_chunks):
    chunks_per_core = n_chunks // NCORES
    return functools.partial(
        pl.kernel,
        out_type=jax.ShapeDtypeStruct((n_chunks, NPAD, C), jnp.float32),
        mesh=_sc_mesh(),
        scratch_types=[
            pltpu.VMEM((NB_MAIN // 2, B), jnp.int32),
            pltpu.VMEM((NB_MAIN // 2, B), jnp.int32),
            pltpu.VMEM((B, C), jnp.float32),
            pltpu.VMEM((B, C), jnp.float32),
            pltpu.SemaphoreType.DMA,
            pltpu.SemaphoreType.DMA,
            pltpu.VMEM_SHARED((NACC, C), jnp.float32),
        ],
    )(_make_scatter_body(chunks_per_core))


RB = 1280
NRB = NPAD // RB
DRB = RB // 128


def _dinv_body(deg_ref, o_ref):
    deg = deg_ref[0, :, :] + deg_ref[1, :, :] + 1.0
    o_ref[...] = lax.rsqrt(deg)


def _tc_dinv(deg2):
    return pl.pallas_call(
        _dinv_body,
        grid=(NPAD // RB,),
        in_specs=[pl.BlockSpec((NCORES, RB, C), lambda i: (0, i, 0))],
        out_specs=pl.BlockSpec((RB, 128), lambda i: (i, 0)),
        out_shape=jax.ShapeDtypeStruct((NPAD, 128), jnp.float32),
    )(deg2)


def _mm1_body(dinv_ref, x_ref, w_ref, o_ref):
    dinv = dinv_ref[:, 0:1]
    xw = jnp.dot(x_ref[...], w_ref[...], preferred_element_type=jnp.float32)
    y = dinv * xw
    for k in range(D_HID // C):
        o_ref[k] = y[:, k * C:(k + 1) * C]


def _tc_mm1(dinv_c, x, w1):
    n_chunks = w1.shape[1] // C
    return pl.pallas_call(
        _mm1_body,
        grid=(NPAD // RB,),
        in_specs=[
            pl.BlockSpec((RB, 128), lambda i: (i, 0)),
            pl.BlockSpec((RB, D_IN), lambda i: (i, 0)),
            pl.BlockSpec((D_IN, D_HID), lambda i: (0, 0)),
        ],
        out_specs=pl.BlockSpec((n_chunks, RB, C), lambda i: (0, i, 0)),
        out_shape=jax.ShapeDtypeStruct((n_chunks, NPAD, C), jnp.float32),
    )(dinv_c, x, w1)


def _mm2_body(dinv_ref, s1_ref, b1_ref, w_ref, o_ref):
    dinv = dinv_ref[:, 0:1]
    s1 = jnp.concatenate([s1_ref[k] for k in range(4)], axis=1)
    h = jax.nn.relu(dinv * s1 + b1_ref[0][None, :])
    y = dinv * jnp.dot(h, w_ref[...], preferred_element_type=jnp.float32)
    for k in range(D_OUT // C):
        o_ref[k] = y[:, k * C:(k + 1) * C]


def _tc_mm2(dinv_c, s1, b1, w2):
    n1 = s1.shape[0]
    n_chunks = w2.shape[1] // C
    return pl.pallas_call(
        _mm2_body,
        grid=(NPAD // RB,),
        in_specs=[
            pl.BlockSpec((RB, 128), lambda i: (i, 0)),
            pl.BlockSpec((n1, RB, C), lambda i: (0, i, 0)),
            pl.BlockSpec((1, D_HID), lambda i: (0, 0)),
            pl.BlockSpec((D_HID, D_OUT), lambda i: (0, 0)),
        ],
        out_specs=pl.BlockSpec((n_chunks, RB, C), lambda i: (0, i, 0)),
        out_shape=jax.ShapeDtypeStruct((n_chunks, NPAD, C), jnp.float32),
    )(dinv_c, s1, b1, w2)


def _fin_body(dinv_ref, s2_ref, b2_ref, o_ref):
    dinv = dinv_ref[:, 0:1]
    s2 = jnp.concatenate([s2_ref[k] for k in range(2)], axis=1)
    o_ref[...] = dinv * s2 + b2_ref[0][None, :]


def _tc_final(dinv_c, s2, b2):
    n2 = s2.shape[0]
    return pl.pallas_call(
        _fin_body,
        grid=(NPAD // RB,),
        in_specs=[
            pl.BlockSpec((RB, 128), lambda i: (i, 0)),
            pl.BlockSpec((n2, RB, C), lambda i: (0, i, 0)),
            pl.BlockSpec((1, D_OUT), lambda i: (0, 0)),
        ],
        out_specs=pl.BlockSpec((RB, D_OUT), lambda i: (i, 0)),
        out_shape=jax.ShapeDtypeStruct((NPAD, D_OUT), jnp.float32),
    )(dinv_c, s2, b2)


def _pad_scatter_rows(n):
    return N + (jnp.arange(n, dtype=jnp.int32) % (NACC - N))


def _pad_indices(src, dst):
    pad = E_MAIN - E
    src_p = jnp.concatenate([src, jnp.arange(pad, dtype=jnp.int32) % N])
    dst_p = jnp.concatenate([dst, _pad_scatter_rows(pad)])
    return (src_p.reshape(NTILES, NB_MAIN, B),
            dst_p.reshape(NTILES, NB_MAIN, B))


def _pad_deg_indices(dst):
    pad = E_DEG - E
    dst_p = jnp.concatenate([dst, _pad_scatter_rows(pad)])
    return dst_p.reshape(NCORES, NTILES, NB_DEG, B)


@jax.jit
def kernel(x, edge_index, W1, b1, W2, b2):
    src = edge_index[0].astype(jnp.int32)
    dst = edge_index[1].astype(jnp.int32)
    src_m, dst_m = _pad_indices(src, dst)
    dst_d = _pad_deg_indices(dst)

    ones = jnp.ones((B, C), jnp.float32)
    zeros = jnp.zeros((NACC // NTILES, C), jnp.float32)

    deg2 = _make_deg_kernel()(dst_d, ones, zeros)
    dinv_c = _tc_dinv(deg2)

    x_p = jnp.zeros((NPAD, D_IN), jnp.float32).at[:N].set(x)
    y1 = _tc_mm1(dinv_c, x_p, W1)
    s1 = _make_scatter_kernel(D_HID // C)(y1, src_m, dst_m)
    y2 = _tc_mm2(dinv_c, s1, b1.reshape(1, D_HID), W2)
    s2 = _make_scatter_kernel(D_OUT // C)(y2, src_m, dst_m)
    out = _tc_final(dinv_c, s2, b2.reshape(1, D_OUT))
    return out[:N]


# speedup vs baseline: 2.7291x; 1.0113x over previous
import functools

import jax
import jax.numpy as jnp
from jax import lax
from jax.experimental import pallas as pl
from jax.experimental.pallas import tpu as pltpu
from jax.experimental.pallas import tpu_sc as plsc

N = 10000
E = 160000
D_IN = 256
D_HID = 512
D_OUT = 256

C = 128
B = 128
NTILES = 16
NCORES = 2
NACC = 10240
PAD_DST = N + 8

NB_MAIN = 80
E_MAIN = NTILES * NB_MAIN * B

NB_DEG = 40
E_DEG = NCORES * NTILES * NB_DEG * B

NPAD = NACC
ROWS_PER_TILE = NPAD // NTILES


def _sc_mesh():
    return plsc.VectorSubcoreMesh(core_axis_name="c", subcore_axis_name="s")


def _deg_body(dst_hbm, ones_hbm, zeros_hbm, out_hbm, idx_v, ones_v, sem, acc):
    c = lax.axis_index("c")
    s = lax.axis_index("s")
    pltpu.sync_copy(dst_hbm.at[c].at[s], idx_v)
    pltpu.sync_copy(ones_hbm, ones_v)
    z0 = s * (NACC // NTILES)
    pltpu.sync_copy(zeros_hbm, acc.at[pl.ds(z0, NACC // NTILES)])
    plsc.subcore_barrier()

    def step(j, carry):
        pltpu.sync_copy(ones_v, acc.at[idx_v.at[j]], add=True)
        return carry

    lax.fori_loop(0, NB_DEG, step, 0)
    plsc.subcore_barrier()
    r0 = s * (NACC // NTILES)
    pltpu.sync_copy(acc.at[pl.ds(r0, NACC // NTILES)],
                    out_hbm.at[c].at[pl.ds(r0, NACC // NTILES)])


def _make_deg_kernel():
    return functools.partial(
        pl.kernel,
        out_type=jax.ShapeDtypeStruct((NCORES, NACC, C), jnp.float32),
        mesh=_sc_mesh(),
        scratch_types=[
            pltpu.VMEM((NB_DEG, B), jnp.int32),
            pltpu.VMEM((B, C), jnp.float32),
            pltpu.SemaphoreType.DMA,
            pltpu.VMEM_SHARED((NACC, C), jnp.float32),
        ],
    )(_deg_body)


def _make_scatter_body(chunks_per_core):
    def body(y_hbm, src_hbm, dst_hbm, out_hbm, src_v, dst_v,
             buf0, buf1, sg0, sg1, acc):
        c = lax.axis_index("c")
        s = lax.axis_index("s")
        r0 = s * ROWS_PER_TILE
        HNB = NB_MAIN // 2
        H = HNB // 2
        for cc_local in range(chunks_per_core):
            cc = c * chunks_per_core + cc_local
            yc = y_hbm.at[cc]
            pltpu.sync_copy(yc.at[pl.ds(r0, ROWS_PER_TILE)],
                            acc.at[pl.ds(r0, ROWS_PER_TILE)])
            plsc.subcore_barrier()

            for half in range(2):
                pltpu.sync_copy(src_hbm.at[s].at[pl.ds(half * HNB, HNB)], src_v)
                pltpu.sync_copy(dst_hbm.at[s].at[pl.ds(half * HNB, HNB)], dst_v)
                pltpu.async_copy(yc.at[src_v.at[0]], buf0, sg0)

                def step(p, carry):
                    j0 = 2 * p
                    pltpu.async_copy(yc.at[src_v.at[j0 + 1]], buf1, sg1)
                    pltpu.make_async_copy(yc.at[src_v.at[j0]], buf0, sg0).wait()
                    pltpu.sync_copy(buf0, acc.at[dst_v.at[j0]], add=True)

                    @pl.when(p < H - 1)
                    def _():
                        pltpu.async_copy(yc.at[src_v.at[j0 + 2]], buf0, sg0)

                    pltpu.make_async_copy(yc.at[src_v.at[j0 + 1]],
                                          buf1, sg1).wait()
                    pltpu.sync_copy(buf1, acc.at[dst_v.at[j0 + 1]], add=True)
                    return carry

                lax.fori_loop(0, H, step, 0)
            plsc.subcore_barrier()
            pltpu.sync_copy(acc.at[pl.ds(r0, ROWS_PER_TILE)],
                            out_hbm.at[cc].at[pl.ds(r0, ROWS_PER_TILE)])
            plsc.subcore_barrier()

    return body


def _make_scatter_kernel(n_chunks):
    chunks_per_core = n_chunks // NCORES
    return functools.partial(
        pl.kernel,
        out_type=jax.ShapeDtypeStruct((n_chunks, NPAD, C), jnp.float32),
        mesh=_sc_mesh(),
        scratch_types=[
            pltpu.VMEM((NB_MAIN // 2, B), jnp.int32),
            pltpu.VMEM((NB_MAIN // 2, B), jnp.int32),
            pltpu.VMEM((B, C), jnp.float32),
            pltpu.VMEM((B, C), jnp.float32),
            pltpu.SemaphoreType.DMA,
            pltpu.SemaphoreType.DMA,
            pltpu.VMEM_SHARED((NACC, C), jnp.float32),
        ],
    )(_make_scatter_body(chunks_per_core))


RB = 2560
NRB = NPAD // RB
DRB = RB // 128


def _dinv_body(deg_ref, o_ref):
    deg = deg_ref[0, :, :] + deg_ref[1, :, :] + 1.0
    o_ref[...] = lax.rsqrt(deg)


def _tc_dinv(deg2):
    return pl.pallas_call(
        _dinv_body,
        grid=(NPAD // RB,),
        in_specs=[pl.BlockSpec((NCORES, RB, C), lambda i: (0, i, 0))],
        out_specs=pl.BlockSpec((RB, 128), lambda i: (i, 0)),
        out_shape=jax.ShapeDtypeStruct((NPAD, 128), jnp.float32),
    )(deg2)


def _mm1_body(dinv_ref, x_ref, w_ref, o_ref):
    dinv = dinv_ref[:, 0:1]
    xw = jnp.dot(x_ref[...], w_ref[...], preferred_element_type=jnp.float32)
    y = dinv * xw
    for k in range(D_HID // C):
        o_ref[k] = y[:, k * C:(k + 1) * C]


def _tc_mm1(dinv_c, x, w1):
    n_chunks = w1.shape[1] // C
    return pl.pallas_call(
        _mm1_body,
        grid=(NPAD // RB,),
        in_specs=[
            pl.BlockSpec((RB, 128), lambda i: (i, 0)),
            pl.BlockSpec((RB, D_IN), lambda i: (i, 0)),
            pl.BlockSpec((D_IN, D_HID), lambda i: (0, 0)),
        ],
        out_specs=pl.BlockSpec((n_chunks, RB, C), lambda i: (0, i, 0)),
        out_shape=jax.ShapeDtypeStruct((n_chunks, NPAD, C), jnp.float32),
    )(dinv_c, x, w1)


def _mm2_body(dinv_ref, s1_ref, b1_ref, w_ref, o_ref):
    dinv = dinv_ref[:, 0:1]
    s1 = jnp.concatenate([s1_ref[k] for k in range(4)], axis=1)
    h = jax.nn.relu(dinv * s1 + b1_ref[0][None, :])
    y = dinv * jnp.dot(h, w_ref[...], preferred_element_type=jnp.float32)
    for k in range(D_OUT // C):
        o_ref[k] = y[:, k * C:(k + 1) * C]


def _tc_mm2(dinv_c, s1, b1, w2):
    n1 = s1.shape[0]
    n_chunks = w2.shape[1] // C
    return pl.pallas_call(
        _mm2_body,
        grid=(NPAD // RB,),
        in_specs=[
            pl.BlockSpec((RB, 128), lambda i: (i, 0)),
            pl.BlockSpec((n1, RB, C), lambda i: (0, i, 0)),
            pl.BlockSpec((1, D_HID), lambda i: (0, 0)),
            pl.BlockSpec((D_HID, D_OUT), lambda i: (0, 0)),
        ],
        out_specs=pl.BlockSpec((n_chunks, RB, C), lambda i: (0, i, 0)),
        out_shape=jax.ShapeDtypeStruct((n_chunks, NPAD, C), jnp.float32),
    )(dinv_c, s1, b1, w2)


def _fin_body(dinv_ref, s2_ref, b2_ref, o_ref):
    dinv = dinv_ref[:, 0:1]
    s2 = jnp.concatenate([s2_ref[k] for k in range(2)], axis=1)
    o_ref[...] = dinv * s2 + b2_ref[0][None, :]


def _tc_final(dinv_c, s2, b2):
    n2 = s2.shape[0]
    return pl.pallas_call(
        _fin_body,
        grid=(NPAD // RB,),
        in_specs=[
            pl.BlockSpec((RB, 128), lambda i: (i, 0)),
            pl.BlockSpec((n2, RB, C), lambda i: (0, i, 0)),
            pl.BlockSpec((1, D_OUT), lambda i: (0, 0)),
        ],
        out_specs=pl.BlockSpec((RB, D_OUT), lambda i: (i, 0)),
        out_shape=jax.ShapeDtypeStruct((NPAD, D_OUT), jnp.float32),
    )(dinv_c, s2, b2)


def _pad_scatter_rows(n):
    return N + (jnp.arange(n, dtype=jnp.int32) % (NACC - N))


def _pad_indices(src, dst):
    pad = E_MAIN - E
    src_p = jnp.concatenate([src, jnp.arange(pad, dtype=jnp.int32) % N])
    dst_p = jnp.concatenate([dst, _pad_scatter_rows(pad)])
    return (src_p.reshape(NTILES, NB_MAIN, B),
            dst_p.reshape(NTILES, NB_MAIN, B))


def _pad_deg_indices(dst):
    pad = E_DEG - E
    dst_p = jnp.concatenate([dst, _pad_scatter_rows(pad)])
    return dst_p.reshape(NCORES, NTILES, NB_DEG, B)


@jax.jit
def kernel(x, edge_index, W1, b1, W2, b2):
    src = edge_index[0].astype(jnp.int32)
    dst = edge_index[1].astype(jnp.int32)
    src_m, dst_m = _pad_indices(src, dst)
    dst_d = _pad_deg_indices(dst)

    ones = jnp.ones((B, C), jnp.float32)
    zeros = jnp.zeros((NACC // NTILES, C), jnp.float32)

    deg2 = _make_deg_kernel()(dst_d, ones, zeros)
    dinv_c = _tc_dinv(deg2)

    x_p = jnp.zeros((NPAD, D_IN), jnp.float32).at[:N].set(x)
    y1 = _tc_mm1(dinv_c, x_p, W1)
    s1 = _make_scatter_kernel(D_HID // C)(y1, src_m, dst_m)
    y2 = _tc_mm2(dinv_c, s1, b1.reshape(1, D_HID), W2)
    s2 = _make_scatter_kernel(D_OUT // C)(y2, src_m, dst_m)
    out = _tc_final(dinv_c, s2, b2.reshape(1, D_OUT))
    return out[:N]


# speedup vs baseline: 2.7463x; 1.0063x over previous
import functools

import jax
import jax.numpy as jnp
from jax import lax
from jax.experimental import pallas as pl
from jax.experimental.pallas import tpu as pltpu
from jax.experimental.pallas import tpu_sc as plsc

N = 10000
E = 160000
D_IN = 256
D_HID = 512
D_OUT = 256

C = 128
B = 128
NTILES = 16
NCORES = 2
NACC = 10240
PAD_DST = N + 8

NB_MAIN = 80
E_MAIN = NTILES * NB_MAIN * B

NB_DEG = 40
E_DEG = NCORES * NTILES * NB_DEG * B

NPAD = NACC
ROWS_PER_TILE = NPAD // NTILES


def _sc_mesh():
    return plsc.VectorSubcoreMesh(core_axis_name="c", subcore_axis_name="s")


def _deg_body(dst_hbm, ones_hbm, zeros_hbm, out_hbm, idx_v, ones_v, sem, acc):
    c = lax.axis_index("c")
    s = lax.axis_index("s")
    pltpu.sync_copy(dst_hbm.at[c].at[s], idx_v)
    pltpu.sync_copy(ones_hbm, ones_v)
    z0 = s * (NACC // NTILES)
    pltpu.sync_copy(zeros_hbm, acc.at[pl.ds(z0, NACC // NTILES)])
    plsc.subcore_barrier()

    def step(j, carry):
        pltpu.sync_copy(ones_v, acc.at[idx_v.at[j]], add=True)
        return carry

    lax.fori_loop(0, NB_DEG, step, 0)
    plsc.subcore_barrier()
    r0 = s * (NACC // NTILES)
    pltpu.sync_copy(acc.at[pl.ds(r0, NACC // NTILES)],
                    out_hbm.at[c].at[pl.ds(r0, NACC // NTILES)])


def _make_deg_kernel():
    return functools.partial(
        pl.kernel,
        out_type=jax.ShapeDtypeStruct((NCORES, NACC, C), jnp.float32),
        mesh=_sc_mesh(),
        scratch_types=[
            pltpu.VMEM((NB_DEG, B), jnp.int32),
            pltpu.VMEM((B, C), jnp.float32),
            pltpu.SemaphoreType.DMA,
            pltpu.VMEM_SHARED((NACC, C), jnp.float32),
        ],
    )(_deg_body)


def _make_scatter_body(chunks_per_core):
    def body(y_hbm, src_hbm, dst_hbm, out_hbm, src_v, dst_v,
             buf0, buf1, sg0, sg1, acc):
        c = lax.axis_index("c")
        s = lax.axis_index("s")
        r0 = s * ROWS_PER_TILE
        HNB = NB_MAIN // 2
        H = HNB // 2
        for cc_local in range(chunks_per_core):
            cc = c * chunks_per_core + cc_local
            yc = y_hbm.at[cc]
            pltpu.sync_copy(yc.at[pl.ds(r0, ROWS_PER_TILE)],
                            acc.at[pl.ds(r0, ROWS_PER_TILE)])
            plsc.subcore_barrier()

            for half in range(2):
                pltpu.sync_copy(src_hbm.at[s].at[pl.ds(half * HNB, HNB)], src_v)
                pltpu.sync_copy(dst_hbm.at[s].at[pl.ds(half * HNB, HNB)], dst_v)
                pltpu.async_copy(yc.at[src_v.at[0]], buf0, sg0)

                def step(p, carry):
                    j0 = 2 * p
                    pltpu.async_copy(yc.at[src_v.at[j0 + 1]], buf1, sg1)
                    pltpu.make_async_copy(yc.at[src_v.at[j0]], buf0, sg0).wait()
                    pltpu.sync_copy(buf0, acc.at[dst_v.at[j0]], add=True)

                    @pl.when(p < H - 1)
                    def _():
                        pltpu.async_copy(yc.at[src_v.at[j0 + 2]], buf0, sg0)

                    pltpu.make_async_copy(yc.at[src_v.at[j0 + 1]],
                                          buf1, sg1).wait()
                    pltpu.sync_copy(buf1, acc.at[dst_v.at[j0 + 1]], add=True)
                    return carry

                lax.fori_loop(0, H, step, 0)
            plsc.subcore_barrier()
            pltpu.sync_copy(acc.at[pl.ds(r0, ROWS_PER_TILE)],
                            out_hbm.at[cc].at[pl.ds(r0, ROWS_PER_TILE)])
            plsc.subcore_barrier()

    return body


def _make_scatter_kernel(n_chunks):
    chunks_per_core = n_chunks // NCORES
    return functools.partial(
        pl.kernel,
        out_type=jax.ShapeDtypeStruct((n_chunks, NPAD, C), jnp.float32),
        mesh=_sc_mesh(),
        scratch_types=[
            pltpu.VMEM((NB_MAIN // 2, B), jnp.int32),
            pltpu.VMEM((NB_MAIN // 2, B), jnp.int32),
            pltpu.VMEM((B, C), jnp.float32),
            pltpu.VMEM((B, C), jnp.float32),
            pltpu.SemaphoreType.DMA,
            pltpu.SemaphoreType.DMA,
            pltpu.VMEM_SHARED((NACC, C), jnp.float32),
        ],
    )(_make_scatter_body(chunks_per_core))


RB = 5120
NRB = NPAD // RB
DRB = RB // 128


def _dinv_body(deg_ref, o_ref):
    deg = deg_ref[0, :, :] + deg_ref[1, :, :] + 1.0
    o_ref[...] = lax.rsqrt(deg)


def _tc_dinv(deg2):
    return pl.pallas_call(
        _dinv_body,
        grid=(NPAD // RB,),
        in_specs=[pl.BlockSpec((NCORES, RB, C), lambda i: (0, i, 0))],
        out_specs=pl.BlockSpec((RB, 128), lambda i: (i, 0)),
        out_shape=jax.ShapeDtypeStruct((NPAD, 128), jnp.float32),
    )(deg2)


def _mm1_body(dinv_ref, x_ref, w_ref, o_ref):
    dinv = dinv_ref[:, 0:1]
    xw = jnp.dot(x_ref[...], w_ref[...], preferred_element_type=jnp.float32)
    y = dinv * xw
    for k in range(D_HID // C):
        o_ref[k] = y[:, k * C:(k + 1) * C]


def _tc_mm1(dinv_c, x, w1):
    n_chunks = w1.shape[1] // C
    return pl.pallas_call(
        _mm1_body,
        grid=(NPAD // RB,),
        in_specs=[
            pl.BlockSpec((RB, 128), lambda i: (i, 0)),
            pl.BlockSpec((RB, D_IN), lambda i: (i, 0)),
            pl.BlockSpec((D_IN, D_HID), lambda i: (0, 0)),
        ],
        out_specs=pl.BlockSpec((n_chunks, RB, C), lambda i: (0, i, 0)),
        out_shape=jax.ShapeDtypeStruct((n_chunks, NPAD, C), jnp.float32),
    )(dinv_c, x, w1)


def _mm2_body(dinv_ref, s1_ref, b1_ref, w_ref, o_ref):
    dinv = dinv_ref[:, 0:1]
    s1 = jnp.concatenate([s1_ref[k] for k in range(4)], axis=1)
    h = jax.nn.relu(dinv * s1 + b1_ref[0][None, :])
    y = dinv * jnp.dot(h, w_ref[...], preferred_element_type=jnp.float32)
    for k in range(D_OUT // C):
        o_ref[k] = y[:, k * C:(k + 1) * C]


def _tc_mm2(dinv_c, s1, b1, w2):
    n1 = s1.shape[0]
    n_chunks = w2.shape[1] // C
    return pl.pallas_call(
        _mm2_body,
        grid=(NPAD // RB,),
        in_specs=[
            pl.BlockSpec((RB, 128), lambda i: (i, 0)),
            pl.BlockSpec((n1, RB, C), lambda i: (0, i, 0)),
            pl.BlockSpec((1, D_HID), lambda i: (0, 0)),
            pl.BlockSpec((D_HID, D_OUT), lambda i: (0, 0)),
        ],
        out_specs=pl.BlockSpec((n_chunks, RB, C), lambda i: (0, i, 0)),
        out_shape=jax.ShapeDtypeStruct((n_chunks, NPAD, C), jnp.float32),
    )(dinv_c, s1, b1, w2)


def _fin_body(dinv_ref, s2_ref, b2_ref, o_ref):
    dinv = dinv_ref[:, 0:1]
    s2 = jnp.concatenate([s2_ref[k] for k in range(2)], axis=1)
    o_ref[...] = dinv * s2 + b2_ref[0][None, :]


def _tc_final(dinv_c, s2, b2):
    n2 = s2.shape[0]
    return pl.pallas_call(
        _fin_body,
        grid=(NPAD // RB,),
        in_specs=[
            pl.BlockSpec((RB, 128), lambda i: (i, 0)),
            pl.BlockSpec((n2, RB, C), lambda i: (0, i, 0)),
            pl.BlockSpec((1, D_OUT), lambda i: (0, 0)),
        ],
        out_specs=pl.BlockSpec((RB, D_OUT), lambda i: (i, 0)),
        out_shape=jax.ShapeDtypeStruct((NPAD, D_OUT), jnp.float32),
    )(dinv_c, s2, b2)


def _pad_scatter_rows(n):
    return N + (jnp.arange(n, dtype=jnp.int32) % (NACC - N))


def _pad_indices(src, dst):
    pad = E_MAIN - E
    src_p = jnp.concatenate([src, jnp.arange(pad, dtype=jnp.int32) % N])
    dst_p = jnp.concatenate([dst, _pad_scatter_rows(pad)])
    return (src_p.reshape(NTILES, NB_MAIN, B),
            dst_p.reshape(NTILES, NB_MAIN, B))


def _pad_deg_indices(dst):
    pad = E_DEG - E
    dst_p = jnp.concatenate([dst, _pad_scatter_rows(pad)])
    return dst_p.reshape(NCORES, NTILES, NB_DEG, B)


@jax.jit
def kernel(x, edge_index, W1, b1, W2, b2):
    src = edge_index[0].astype(jnp.int32)
    dst = edge_index[1].astype(jnp.int32)
    src_m, dst_m = _pad_indices(src, dst)
    dst_d = _pad_deg_indices(dst)

    ones = jnp.ones((B, C), jnp.float32)
    zeros = jnp.zeros((NACC // NTILES, C), jnp.float32)

    deg2 = _make_deg_kernel()(dst_d, ones, zeros)
    dinv_c = _tc_dinv(deg2)

    x_p = jnp.zeros((NPAD, D_IN), jnp.float32).at[:N].set(x)
    y1 = _tc_mm1(dinv_c, x_p, W1)
    s1 = _make_scatter_kernel(D_HID // C)(y1, src_m, dst_m)
    y2 = _tc_mm2(dinv_c, s1, b1.reshape(1, D_HID), W2)
    s2 = _make_scatter_kernel(D_OUT // C)(y2, src_m, dst_m)
    out = _tc_final(dinv_c, s2, b2.reshape(1, D_OUT))
    return out[:N]
